# baseline (speedup 1.0000x reference)
import functools

import jax
import jax.numpy as jnp
from jax import lax
from jax.experimental import pallas as pl
from jax.experimental.pallas import tpu as pltpu
from jax.experimental.pallas import tpu_sc as plsc

B = 1024
D = 512
H = 256
Z = 128
C = 32
N = 64
K = 8192
BN = B * N

_MB = 256
_RB = 256
_KT = 512

_NC_SC = 2
_NS_SC = 16
_NW = _NC_SC * _NS_SC
_BPW = BN // _NW
_CHUNK = 128
_NCH = _BPW // _CHUNK


def _bdot(a, b):
    return jnp.dot(a, b, preferred_element_type=jnp.float32)


def _encoder_body(x_ref, w1_ref, b1_ref, w2_ref, b2_ref, w3_ref, b3_ref,
                  wp_ref, bp_ref, out_ref):
    h = jnp.maximum(_bdot(x_ref[...], w1_ref[...]) + b1_ref[...], 0.0)
    h = jnp.maximum(_bdot(h, w2_ref[...]) + b2_ref[...], 0.0)
    h = jnp.maximum(_bdot(h, w3_ref[...]) + b3_ref[...], 0.0)
    out_ref[...] = _bdot(h, wp_ref[...]) + bp_ref[...]


def _encoder(x, W1, b1, W2, b2, W3, b3, Wpp, bpp, interpret=False):
    full = lambda *s: pl.BlockSpec(s, lambda i: (0,) * len(s))
    return pl.pallas_call(
        _encoder_body,
        grid=(B // _MB,),
        in_specs=[
            pl.BlockSpec((_MB, D), lambda i: (i, 0)),
            full(D, H), full(1, H),
            full(H, H), full(1, H),
            full(H, Z), full(1, Z),
            full(Z, N * C), full(1, N * C),
        ],
        out_specs=pl.BlockSpec((_MB, N * C), lambda i: (i, 0)),
        out_shape=jax.ShapeDtypeStruct((B, N * C), jnp.float32),
        interpret=interpret,
    )(x, W1, b1.reshape(1, H), W2, b2.reshape(1, H), W3, b3.reshape(1, Z),
      Wpp, bpp.reshape(1, N * C))


_T = K // _KT


def _argmin_body(flat_ref, cbt_ref, idx_ref, bv_ref, bi_ref):
    t = pl.program_id(1)

    @pl.when(t == 0)
    def _init():
        bv_ref[...] = jnp.full((_RB,), jnp.inf, jnp.float32)
        bi_ref[...] = jnp.zeros((_RB,), jnp.int32)

    fb = flat_ref[...]
    cbt = cbt_ref[...]
    cn = jnp.sum(cbt * cbt, axis=0)
    xn = jnp.sum(fb * fb, axis=1)
    s = (xn[:, None] + cn[None, :]) - 2.0 * _bdot(fb, cbt)
    v = jnp.min(s, axis=1)
    iota = lax.broadcasted_iota(jnp.int32, (_RB, _KT), 1)
    i = jnp.min(jnp.where(s == v[:, None], iota, K), axis=1) + t * _KT
    take = v < bv_ref[...]
    bv_ref[...] = jnp.where(take, v, bv_ref[...])
    bi = jnp.where(take, i, bi_ref[...])
    bi_ref[...] = bi

    @pl.when(t == _T - 1)
    def _emit():
        idx_ref[...] = bi


def _vq_argmin(flat, cbT, interpret=False):
    return pl.pallas_call(
        _argmin_body,
        grid=(BN // _RB, _T),
        in_specs=[
            pl.BlockSpec((_RB, C), lambda i, t: (i, 0)),
            pl.BlockSpec((C, _KT), lambda i, t: (0, t)),
        ],
        out_specs=pl.BlockSpec((_RB,), lambda i, t: (i,)),
        out_shape=jax.ShapeDtypeStruct((BN,), jnp.int32),
        scratch_shapes=[
            pltpu.VMEM((_RB,), jnp.float32),
            pltpu.VMEM((_RB,), jnp.int32),
        ],
        compiler_params=pltpu.CompilerParams(
            dimension_semantics=("arbitrary", "arbitrary")),
        interpret=interpret,
    )(flat, cbT)


def _codebook_gather(idx2, codebook):
    mesh = plsc.VectorSubcoreMesh(core_axis_name="c", subcore_axis_name="s")

    @functools.partial(
        pl.kernel,
        mesh=mesh,
        compiler_params=pltpu.CompilerParams(use_tc_tiling_on_sc=False),
        out_type=jax.ShapeDtypeStruct((BN, C), jnp.float32),
        scratch_types=[
            pltpu.VMEM((_NCH, _CHUNK), jnp.int32),
            pltpu.VMEM((_BPW, C), jnp.float32),
            pltpu.SemaphoreType.DMA,
        ],
    )
    def gk(idx_hbm, table_hbm, out_hbm, idx_v, rows_v, sem):
        wid = lax.axis_index("s") * _NC_SC + lax.axis_index("c")
        pltpu.sync_copy(idx_hbm.at[pl.ds(wid * _NCH, _NCH), :], idx_v)
        copies = [
            pltpu.async_copy(table_hbm.at[idx_v.at[j]],
                             rows_v.at[pl.ds(j * _CHUNK, _CHUNK), :], sem)
            for j in range(_NCH)
        ]
        for cp in copies:
            cp.wait()
        pltpu.sync_copy(rows_v, out_hbm.at[pl.ds(wid * _BPW, _BPW), :])

    return gk(idx2, codebook)


def _decoder_body(q_ref, w0_ref, b0_ref, w1_ref, b1_ref, w2_ref, b2_ref,
                  w3_ref, b3_ref, out_ref):
    f32 = jnp.float32
    d0 = jnp.dot(q_ref[...], w0_ref[...], preferred_element_type=f32) + b0_ref[...]
    d1 = jnp.maximum(
        jnp.dot(d0, w1_ref[...], preferred_element_type=f32) + b1_ref[...], 0.0)
    d2 = jnp.maximum(
        jnp.dot(d1, w2_ref[...], preferred_element_type=f32) + b2_ref[...], 0.0)
    out_ref[...] = jnp.dot(d2, w3_ref[...], preferred_element_type=f32) + b3_ref[...]


def _decoder(q, Wd0p, bd0, Wd1, bd1, Wd2, bd2, Wd3, bd3, interpret=False):
    full = lambda *s: pl.BlockSpec(s, lambda i: (0,) * len(s))
    return pl.pallas_call(
        _decoder_body,
        grid=(B // _MB,),
        in_specs=[
            pl.BlockSpec((_MB, N * C), lambda i: (i, 0)),
            full(N * C, Z), full(1, Z),
            full(Z, H), full(1, H),
            full(H, H), full(1, H),
            full(H, D), full(1, D),
        ],
        out_specs=pl.BlockSpec((_MB, D), lambda i: (i, 0)),
        out_shape=jax.ShapeDtypeStruct((B, D), jnp.float32),
        interpret=interpret,
    )(q, Wd0p, bd0.reshape(1, Z), Wd1, bd1.reshape(1, H), Wd2, bd2.reshape(1, H),
      Wd3, bd3.reshape(1, D))


def kernel(x, W1, b1, W2, b2, W3, b3, Wp, bp, codebook,
           Wd0, bd0, Wd1, bd1, Wd2, bd2, Wd3, bd3):
    Wpp = Wp.reshape(Z, C, N).transpose(0, 2, 1).reshape(Z, N * C)
    bpp = bp.reshape(C, N).T.reshape(N * C)
    Wd0p = Wd0.reshape(C, N, Z).transpose(1, 0, 2).reshape(N * C, Z)

    flat = _encoder(x, W1, b1, W2, b2, W3, b3, Wpp, bpp).reshape(BN, C)
    idx = _vq_argmin(flat, codebook.T)
    qflat = _codebook_gather(idx.reshape(BN // _CHUNK, _CHUNK), codebook)
    q = qflat.reshape(B, N * C)
    return _decoder(q, Wd0p, bd0, Wd1, bd1, Wd2, bd2, Wd3, bd3)
